# baseline (speedup 1.0000x reference)
import jax
import jax.numpy as jnp
from jax.experimental import pallas as pl

B = 1024
USER_DIM = 128
REPR_DIM = 32
CORPUS = 100000
ITEM_DIM = 64
TOP_K = 100

CPAD = 100352
BT = 256
CT = 7168


def _score_body(uf_ref, cpad_ref, wret_ref, wrank_ref, scores_ref, ruser_ref):
    ur = jax.lax.dot_general(
        uf_ref[...], wret_ref[...], (((1,), (0,)), ((), ())),
        preferred_element_type=jnp.float32)
    s = jax.lax.dot_general(
        ur, cpad_ref[...], (((1,), (1,)), ((), ())),
        preferred_element_type=jnp.float32)
    scores_ref[...] = s
    ruser_ref[...] = jax.lax.dot_general(
        uf_ref[...], wrank_ref[...], (((1,), (0,)), ((), ())),
        preferred_element_type=jnp.float32)


def kernel(user_features, corpus_embeddings, W_ret, W_rank_user, item_table):
    cpad = jnp.pad(corpus_embeddings, ((0, CPAD - CORPUS), (0, 0)))
    grid = (B // BT, CPAD // CT)
    scores, rank_user = pl.pallas_call(
        _score_body,
        grid=grid,
        in_specs=[
            pl.BlockSpec((BT, USER_DIM), lambda i, j: (i, 0)),
            pl.BlockSpec((CT, REPR_DIM), lambda i, j: (j, 0)),
            pl.BlockSpec((USER_DIM, REPR_DIM), lambda i, j: (0, 0)),
            pl.BlockSpec((USER_DIM, ITEM_DIM), lambda i, j: (0, 0)),
        ],
        out_specs=[
            pl.BlockSpec((BT, CT), lambda i, j: (i, j)),
            pl.BlockSpec((BT, ITEM_DIM), lambda i, j: (i, 0)),
        ],
        out_shape=[
            jax.ShapeDtypeStruct((B, CPAD), jnp.float32),
            jax.ShapeDtypeStruct((B, ITEM_DIM), jnp.float32),
        ],
    )(user_features, cpad, W_ret, W_rank_user)

    _, top_idx = jax.lax.top_k(scores[:, :CORPUS], TOP_K)
    cand = jnp.take(item_table, top_idx, axis=0)
    return jnp.einsum('bd,bkd->bk', rank_user, cand)
